# Initial kernel scaffold; baseline (speedup 1.0000x reference)
import jax
import jax.numpy as jnp
from jax.experimental import pallas as pl


def kernel(pillar_features, voxel_coords):
    raise NotImplementedError("write your pallas kernel here")


# speedup vs baseline: 7.5630x; 7.5630x over previous
import functools

import jax
import jax.numpy as jnp
from jax import lax
from jax.experimental import pallas as pl
from jax.experimental.pallas import tpu as pltpu
from jax.experimental.pallas import tpu_sc as plsc

NX = 432
NY = 496
C = 64
M = 12000
B = 4
NXY = NX * NY
ROWS_BIG = 32
ROWS_SMALL = 24
NBIG = 14
CELLS_BIG = ROWS_BIG * NX
CELLS_SMALL = ROWS_SMALL * NX
CHUNK = 2000
NCHUNK = M // CHUNK
GC = CHUNK // 16
CAP = M + 16


def _sc_body(pf_hbm, vc_hbm, out_hbm,
             cbuf, winner, cj, cw, row, ob0, ob1, tmp16, sem0, sem1):
    core = lax.axis_index("c")
    sid = lax.axis_index("s")
    big = sid < NBIG
    lo = jnp.where(big, sid * CELLS_BIG,
                   NBIG * CELLS_BIG + (sid - NBIG) * CELLS_SMALL)
    ncell = jnp.where(big, CELLS_BIG, CELLS_SMALL)
    y0 = pl.multiple_of(lo // NX, 8)
    gr = lax.shift_right_logical(ncell, 4)
    iota = lax.iota(jnp.int32, 16)
    zeros16 = jnp.zeros((16,), jnp.int32)

    def out_dma(ob, b, c, sem, do_start, do_wait):
        @pl.when(big)
        def _():
            cp = pltpu.make_async_copy(
                ob, out_hbm.at[b, c, pl.ds(y0, ROWS_BIG)], sem)
            if do_start:
                cp.start()
            if do_wait:
                cp.wait()

        @pl.when(jnp.logical_not(big))
        def _():
            cp = pltpu.make_async_copy(
                ob.at[pl.ds(0, ROWS_SMALL)],
                out_hbm.at[b, c, pl.ds(y0, ROWS_SMALL)], sem)
            if do_start:
                cp.start()
            if do_wait:
                cp.wait()

    for bb in range(2):
        b = core * 2 + bb

        def fill_body(g, _):
            winner[pl.ds(g * 16, 16)] = jnp.full((16,), -1, jnp.int32)
            return 0

        lax.fori_loop(0, gr, fill_body, 0)

        def chunk_body(ch, _):
            off = pl.multiple_of(b * (M * 3) + ch * (CHUNK * 3), 8)
            pltpu.sync_copy(vc_hbm.at[pl.ds(off, CHUNK * 3)], cbuf)

            def g_body(g, _):
                rows = (g * 16 + iota) * 3
                z = plsc.load_gather(cbuf, [rows])
                y = plsc.load_gather(cbuf, [rows + 1])
                x = plsc.load_gather(cbuf, [rows + 2])
                idx = z + y * NX + x
                key = idx * 16 + iota
                m = ch * CHUNK + g * 16 + iota
                ks, vs = plsc.sort_key_val(key, m)
                sidx = lax.shift_right_logical(ks, 4)
                tmp16[:] = sidx
                nxt = plsc.load_gather(tmp16, [jnp.minimum(iota + 1, 15)])
                is_last = (sidx != nxt) | (iota == 15)
                in_rng = (sidx >= lo) & (sidx < lo + ncell)
                msk = is_last & in_rng
                pos = jnp.clip(sidx - lo, 0, CELLS_BIG - 1)
                plsc.store_scatter(winner, [pos], vs, mask=msk)
                return 0

            lax.fori_loop(0, GC, g_body, 0)
            return 0

        lax.fori_loop(0, NCHUNK, chunk_body, 0)

        def comp_body(g, off):
            w = winner[pl.ds(g * 16, 16)]
            msk = w >= 0
            j = g * 16 + iota
            plsc.store_compressed(cj.at[pl.ds(off, 16)], j, mask=msk)
            plsc.store_compressed(cw.at[pl.ds(off, 16)], w, mask=msk)
            return off + jnp.max(plsc.all_reduce_population_count(msk))

        K = lax.fori_loop(0, gr, comp_body, jnp.int32(0))
        nK = lax.shift_right_logical(K + 15, 4)

        def zero_body(r, _):
            zv = jnp.zeros((16,), jnp.float32)

            def zcol(cg, _):
                ob0[r, pl.ds(cg * 16, 16)] = zv
                ob1[r, pl.ds(cg * 16, 16)] = zv
                return 0

            lax.fori_loop(0, NX // 16, zcol, 0)
            return 0

        lax.fori_loop(0, ROWS_BIG, zero_body, 0)

        def pair_body(cc, _):
            for k, (ob, sem) in enumerate(((ob0, sem0), (ob1, sem1))):
                c = cc * 2 + k
                poff = pl.multiple_of((b * C + c) * M, 8)
                pltpu.sync_copy(pf_hbm.at[pl.ds(poff, M)], row)

                @pl.when(cc > 0)
                def _wait_prev():
                    out_dma(ob, b, c, sem, do_start=False, do_wait=True)

                def s_body(g, _):
                    ii = g * 16 + iota
                    msk = ii < K
                    jj = cj[pl.ds(g * 16, 16)]
                    ww = cw[pl.ds(g * 16, 16)]
                    vals = plsc.load_gather(
                        row, [jnp.clip(ww, 0, M - 1)], mask=msk)
                    rr = jj // NX
                    xx = jj - rr * NX
                    plsc.store_scatter(ob, [rr, xx], vals, mask=msk)
                    return 0

                lax.fori_loop(0, nK, s_body, 0)
                out_dma(ob, b, c, sem, do_start=True, do_wait=False)
            return 0

        lax.fori_loop(0, C // 2, pair_body, 0)

        out_dma(ob0, b, 0, sem0, do_start=False, do_wait=True)
        out_dma(ob1, b, 0, sem1, do_start=False, do_wait=True)


@functools.partial(
    pl.kernel,
    out_type=jax.ShapeDtypeStruct((B, C, NY, NX), jnp.float32),
    mesh=plsc.VectorSubcoreMesh(core_axis_name="c", subcore_axis_name="s"),
    compiler_params=pltpu.CompilerParams(needs_layout_passes=False),
    scratch_types=[
        pltpu.VMEM((CHUNK * 3,), jnp.int32),
        pltpu.VMEM((CELLS_BIG,), jnp.int32),
        pltpu.VMEM((CAP,), jnp.int32),
        pltpu.VMEM((CAP,), jnp.int32),
        pltpu.VMEM((M,), jnp.float32),
        pltpu.VMEM((ROWS_BIG, NX), jnp.float32),
        pltpu.VMEM((ROWS_BIG, NX), jnp.float32),
        pltpu.VMEM((16,), jnp.int32),
        pltpu.SemaphoreType.DMA,
        pltpu.SemaphoreType.DMA,
    ],
)
def _pillar_scatter(pf_hbm, vc_hbm, out_hbm, *scratch):
    _sc_body(pf_hbm, vc_hbm, out_hbm, *scratch)


@jax.jit
def kernel(pillar_features, voxel_coords):
    pf = pillar_features.reshape(B * C * M)
    vc = voxel_coords.astype(jnp.int32).reshape(B * M * 3)
    return _pillar_scatter(pf, vc)


# speedup vs baseline: 9.1755x; 1.2132x over previous
import functools

import jax
import jax.numpy as jnp
from jax import lax
from jax.experimental import pallas as pl
from jax.experimental.pallas import tpu as pltpu
from jax.experimental.pallas import tpu_sc as plsc

NX = 432
NY = 496
C = 64
M = 12000
B = 4
NXY = NX * NY
ROWS_BIG = 32
ROWS_SMALL = 24
NBIG = 14
CELLS_BIG = ROWS_BIG * NX
CELLS_SMALL = ROWS_SMALL * NX
CHUNK = 2000
NCHUNK = M // CHUNK
GC = CHUNK // 16
CAP = M + 16


def _sc_body(pf_hbm, vc_hbm, out_hbm,
             cbuf0, cbuf1, winner, cj, cw, row0, row1, ob0, ob1, tmp16,
             sem0, sem1, rs0, rs1, cs0, cs1):
    core = lax.axis_index("c")
    sid = lax.axis_index("s")
    big = sid < NBIG
    lo = jnp.where(big, sid * CELLS_BIG,
                   NBIG * CELLS_BIG + (sid - NBIG) * CELLS_SMALL)
    ncell = jnp.where(big, CELLS_BIG, CELLS_SMALL)
    y0 = pl.multiple_of(lo // NX, 8)
    gr = lax.shift_right_logical(ncell, 4)
    iota = lax.iota(jnp.int32, 16)
    zeros16 = jnp.zeros((16,), jnp.int32)

    def out_dma(ob, b, c, sem, do_start, do_wait):
        @pl.when(big)
        def _():
            cp = pltpu.make_async_copy(
                ob, out_hbm.at[b, c, pl.ds(y0, ROWS_BIG)], sem)
            if do_start:
                cp.start()
            if do_wait:
                cp.wait()

        @pl.when(jnp.logical_not(big))
        def _():
            cp = pltpu.make_async_copy(
                ob.at[pl.ds(0, ROWS_SMALL)],
                out_hbm.at[b, c, pl.ds(y0, ROWS_SMALL)], sem)
            if do_start:
                cp.start()
            if do_wait:
                cp.wait()

    for bb in range(2):
        b = core * 2 + bb

        def fill_body(g, _):
            winner[pl.ds(g * 16, 16)] = jnp.full((16,), -1, jnp.int32)
            return 0

        lax.fori_loop(0, gr, fill_body, 0)

        def coord_dma(ch, cb, cs):
            off = pl.multiple_of(b * (M * 3) + ch * (CHUNK * 3), 8)
            return pltpu.make_async_copy(
                vc_hbm.at[pl.ds(off, CHUNK * 3)], cb, cs)

        coord_dma(0, cbuf0, cs0).start()
        coord_dma(1, cbuf1, cs1).start()
        for chp in range(NCHUNK // 2):
            for kk, (cb, cs) in enumerate(((cbuf0, cs0), (cbuf1, cs1))):
                ch = chp * 2 + kk
                coord_dma(ch, cb, cs).wait()

                def g_body(g, _):
                    rows = (g * 16 + iota) * 3
                    z = plsc.load_gather(cb, [rows])
                    y = plsc.load_gather(cb, [rows + 1])
                    x = plsc.load_gather(cb, [rows + 2])
                    idx = z + y * NX + x
                    key = idx * 16 + iota
                    m = ch * CHUNK + g * 16 + iota
                    ks, vs = plsc.sort_key_val(key, m)
                    sidx = lax.shift_right_logical(ks, 4)
                    tmp16[:] = sidx
                    nxt = plsc.load_gather(tmp16, [jnp.minimum(iota + 1, 15)])
                    is_last = (sidx != nxt) | (iota == 15)
                    in_rng = (sidx >= lo) & (sidx < lo + ncell)
                    msk = is_last & in_rng
                    pos = jnp.clip(sidx - lo, 0, CELLS_BIG - 1)
                    plsc.store_scatter(winner, [pos], vs, mask=msk)
                    return 0

                lax.fori_loop(0, GC, g_body, 0)
                if ch + 2 < NCHUNK:
                    coord_dma(ch + 2, cb, cs).start()

        def comp_body(g, off):
            w = winner[pl.ds(g * 16, 16)]
            msk = w >= 0
            j = g * 16 + iota
            plsc.store_compressed(cj.at[pl.ds(off, 16)], j, mask=msk)
            plsc.store_compressed(cw.at[pl.ds(off, 16)], w, mask=msk)
            return off + jnp.max(plsc.all_reduce_population_count(msk))

        K = lax.fori_loop(0, gr, comp_body, jnp.int32(0))
        nK = lax.shift_right_logical(K + 15, 4)

        def zero_body(r, _):
            zv = jnp.zeros((16,), jnp.float32)

            def zcol(cg, _):
                ob0[r, pl.ds(cg * 16, 16)] = zv
                ob1[r, pl.ds(cg * 16, 16)] = zv
                return 0

            lax.fori_loop(0, NX // 16, zcol, 0)
            return 0

        lax.fori_loop(0, ROWS_BIG, zero_body, 0)

        def row_dma(c, rb, rsem):
            poff = pl.multiple_of((b * C + c) * M, 8)
            return pltpu.make_async_copy(pf_hbm.at[pl.ds(poff, M)], rb, rsem)

        row_dma(0, row0, rs0).start()
        row_dma(1, row1, rs1).start()

        def pair_body(cc, _):
            for k, (ob, sem, rb, rsem) in enumerate(
                    ((ob0, sem0, row0, rs0), (ob1, sem1, row1, rs1))):
                c = cc * 2 + k
                row_dma(c, rb, rsem).wait()

                @pl.when(cc > 0)
                def _wait_prev():
                    out_dma(ob, b, c, sem, do_start=False, do_wait=True)

                def s_body(g, _):
                    ii = g * 16 + iota
                    msk = ii < K
                    jj = cj[pl.ds(g * 16, 16)]
                    ww = cw[pl.ds(g * 16, 16)]
                    vals = plsc.load_gather(
                        rb, [jnp.clip(ww, 0, M - 1)], mask=msk)
                    rr = jj // NX
                    xx = jj - rr * NX
                    plsc.store_scatter(ob, [rr, xx], vals, mask=msk)
                    return 0

                lax.fori_loop(0, nK, s_body, 0)
                out_dma(ob, b, c, sem, do_start=True, do_wait=False)

                @pl.when(cc < C // 2 - 1)
                def _prefetch_next():
                    row_dma(c + 2, rb, rsem).start()
            return 0

        lax.fori_loop(0, C // 2, pair_body, 0)

        out_dma(ob0, b, 0, sem0, do_start=False, do_wait=True)
        out_dma(ob1, b, 0, sem1, do_start=False, do_wait=True)


@functools.partial(
    pl.kernel,
    out_type=jax.ShapeDtypeStruct((B, C, NY, NX), jnp.float32),
    mesh=plsc.VectorSubcoreMesh(core_axis_name="c", subcore_axis_name="s"),
    compiler_params=pltpu.CompilerParams(needs_layout_passes=False),
    scratch_types=[
        pltpu.VMEM((CHUNK * 3,), jnp.int32),
        pltpu.VMEM((CHUNK * 3,), jnp.int32),
        pltpu.VMEM((CELLS_BIG,), jnp.int32),
        pltpu.VMEM((CAP,), jnp.int32),
        pltpu.VMEM((CAP,), jnp.int32),
        pltpu.VMEM((M,), jnp.float32),
        pltpu.VMEM((M,), jnp.float32),
        pltpu.VMEM((ROWS_BIG, NX), jnp.float32),
        pltpu.VMEM((ROWS_BIG, NX), jnp.float32),
        pltpu.VMEM((16,), jnp.int32),
        pltpu.SemaphoreType.DMA,
        pltpu.SemaphoreType.DMA,
        pltpu.SemaphoreType.DMA,
        pltpu.SemaphoreType.DMA,
        pltpu.SemaphoreType.DMA,
        pltpu.SemaphoreType.DMA,
    ],
)
def _pillar_scatter(pf_hbm, vc_hbm, out_hbm, *scratch):
    _sc_body(pf_hbm, vc_hbm, out_hbm, *scratch)


@jax.jit
def kernel(pillar_features, voxel_coords):
    pf = pillar_features.reshape(B * C * M)
    vc = voxel_coords.astype(jnp.int32).reshape(B * M * 3)
    return _pillar_scatter(pf, vc)


# speedup vs baseline: 12.7178x; 1.3861x over previous
import functools

import jax
import jax.numpy as jnp
from jax import lax
from jax.experimental import pallas as pl
from jax.experimental.pallas import tpu as pltpu
from jax.experimental.pallas import tpu_sc as plsc

NX = 432
NY = 496
C = 64
M = 12000
B = 4
NXY = NX * NY
ROWS_BIG = 32
ROWS_SMALL = 24
NBIG = 14
CELLS_BIG = ROWS_BIG * NX
CELLS_SMALL = ROWS_SMALL * NX
CHUNK = 2000
NCHUNK = M // CHUNK
GC = CHUNK // 16
CAP = M + 16


def _sc_body(pf_hbm, vc_hbm, out_hbm,
             cbuf0, cbuf1, winner, cjw, row0, row1, ob0, ob1, tmp16,
             sem0, sem1, rs0, rs1, cs0, cs1):
    core = lax.axis_index("c")
    sid = lax.axis_index("s")
    big = sid < NBIG
    lo = jnp.where(big, sid * CELLS_BIG,
                   NBIG * CELLS_BIG + (sid - NBIG) * CELLS_SMALL)
    ncell = jnp.where(big, CELLS_BIG, CELLS_SMALL)
    y0 = pl.multiple_of(lo // NX, 8)
    gr = lax.shift_right_logical(ncell, 4)
    iota = lax.iota(jnp.int32, 16)
    zeros16 = jnp.zeros((16,), jnp.int32)

    def out_dma(ob, b, cpair, sem, do_start, do_wait):
        @pl.when(big)
        def _():
            cp = pltpu.make_async_copy(
                ob,
                out_hbm.at[b, pl.ds(cpair, 2), pl.ds(y0, ROWS_BIG)], sem)
            if do_start:
                cp.start()
            if do_wait:
                cp.wait()

        @pl.when(jnp.logical_not(big))
        def _():
            cp = pltpu.make_async_copy(
                ob.at[:, pl.ds(0, ROWS_SMALL)],
                out_hbm.at[b, pl.ds(cpair, 2), pl.ds(y0, ROWS_SMALL)], sem)
            if do_start:
                cp.start()
            if do_wait:
                cp.wait()

    for bb in range(2):
        b = core * 2 + bb

        def fill_body(g, _):
            winner[pl.ds(g * 16, 16)] = jnp.full((16,), -1, jnp.int32)
            return 0

        lax.fori_loop(0, gr, fill_body, 0)

        def coord_dma(ch, cb, cs):
            off = pl.multiple_of(b * (M * 3) + ch * (CHUNK * 3), 8)
            return pltpu.make_async_copy(
                vc_hbm.at[pl.ds(off, CHUNK * 3)], cb, cs)

        coord_dma(0, cbuf0, cs0).start()
        coord_dma(1, cbuf1, cs1).start()
        for chp in range(NCHUNK // 2):
            for kk, (cb, cs) in enumerate(((cbuf0, cs0), (cbuf1, cs1))):
                ch = chp * 2 + kk
                coord_dma(ch, cb, cs).wait()

                def g_body(g, _):
                    rows = (g * 16 + iota) * 3
                    z = plsc.load_gather(cb, [rows])
                    y = plsc.load_gather(cb, [rows + 1])
                    x = plsc.load_gather(cb, [rows + 2])
                    idx = z + y * NX + x
                    key = idx * 16 + iota
                    m = ch * CHUNK + g * 16 + iota
                    ks, vs = plsc.sort_key_val(key, m)
                    sidx = lax.shift_right_logical(ks, 4)
                    tmp16[:] = sidx
                    nxt = plsc.load_gather(tmp16, [jnp.minimum(iota + 1, 15)])
                    is_last = (sidx != nxt) | (iota == 15)
                    in_rng = (sidx >= lo) & (sidx < lo + ncell)
                    msk = is_last & in_rng
                    pos = jnp.clip(sidx - lo, 0, CELLS_BIG - 1)
                    plsc.store_scatter(winner, [pos], vs, mask=msk)
                    return 0

                lax.fori_loop(0, GC, g_body, 0)
                if ch + 2 < NCHUNK:
                    coord_dma(ch + 2, cb, cs).start()

        def comp_body(g, off):
            w = winner[pl.ds(g * 16, 16)]
            msk = w >= 0
            j = g * 16 + iota
            packed = lax.shift_left(j, 14) | jnp.maximum(w, 0)
            plsc.store_compressed(cjw.at[pl.ds(off, 16)], packed, mask=msk)
            return off + jnp.max(plsc.all_reduce_population_count(msk))

        K = lax.fori_loop(0, gr, comp_body, jnp.int32(0))
        nK = lax.shift_right_logical(K + 15, 4)

        def zero_body(r, _):
            zv = jnp.zeros((16,), jnp.float32)

            def zcol(cg, _):
                for ob in (ob0, ob1):
                    ob[0, r, pl.ds(cg * 16, 16)] = zv
                    ob[1, r, pl.ds(cg * 16, 16)] = zv
                return 0

            lax.fori_loop(0, NX // 16, zcol, 0)
            return 0

        lax.fori_loop(0, ROWS_BIG, zero_body, 0)

        def row_dma(c, rb, rsem):
            poff = pl.multiple_of((b * C + c) * M, 8)
            return pltpu.make_async_copy(pf_hbm.at[pl.ds(poff, M)], rb, rsem)

        row_dma(0, row0, rs0).start()
        row_dma(1, row1, rs1).start()

        def quad_body(q, _):
            for k, (ob, sem) in enumerate(((ob0, sem0), (ob1, sem1))):
                cpair = q * 4 + 2 * k

                @pl.when(q > 0)
                def _wait_prev():
                    out_dma(ob, b, cpair, sem, do_start=False, do_wait=True)

                for h, (rb, rsem) in enumerate(((row0, rs0), (row1, rs1))):
                    c = cpair + h
                    row_dma(c, rb, rsem).wait()

                    def s_body(g, _):
                        ii = g * 16 + iota
                        msk = ii < K
                        pk = cjw[pl.ds(g * 16, 16)]
                        ww = jnp.minimum(pk & 0x3FFF, M - 1)
                        jj = jnp.minimum(
                            lax.shift_right_logical(pk, 14), CELLS_BIG - 1)
                        vals = plsc.load_gather(rb, [ww], mask=msk)
                        rr = jj // NX
                        xx = jj - rr * NX
                        plsc.store_scatter(
                            ob, [zeros16 + h, rr, xx], vals, mask=msk)
                        return 0

                    lax.fori_loop(0, nK, s_body, 0)

                    @pl.when(c + 2 < C)
                    def _prefetch_next():
                        row_dma(c + 2, rb, rsem).start()

                out_dma(ob, b, cpair, sem, do_start=True, do_wait=False)
            return 0

        lax.fori_loop(0, C // 4, quad_body, 0)

        out_dma(ob0, b, 0, sem0, do_start=False, do_wait=True)
        out_dma(ob1, b, 0, sem1, do_start=False, do_wait=True)


@functools.partial(
    pl.kernel,
    out_type=jax.ShapeDtypeStruct((B, C, NY, NX), jnp.float32),
    mesh=plsc.VectorSubcoreMesh(core_axis_name="c", subcore_axis_name="s"),
    compiler_params=pltpu.CompilerParams(needs_layout_passes=False),
    scratch_types=[
        pltpu.VMEM((CHUNK * 3,), jnp.int32),
        pltpu.VMEM((CHUNK * 3,), jnp.int32),
        pltpu.VMEM((CELLS_BIG,), jnp.int32),
        pltpu.VMEM((CAP,), jnp.int32),
        pltpu.VMEM((M,), jnp.float32),
        pltpu.VMEM((M,), jnp.float32),
        pltpu.VMEM((2, ROWS_BIG, NX), jnp.float32),
        pltpu.VMEM((2, ROWS_BIG, NX), jnp.float32),
        pltpu.VMEM((16,), jnp.int32),
        pltpu.SemaphoreType.DMA,
        pltpu.SemaphoreType.DMA,
        pltpu.SemaphoreType.DMA,
        pltpu.SemaphoreType.DMA,
        pltpu.SemaphoreType.DMA,
        pltpu.SemaphoreType.DMA,
    ],
)
def _pillar_scatter(pf_hbm, vc_hbm, out_hbm, *scratch):
    _sc_body(pf_hbm, vc_hbm, out_hbm, *scratch)


@jax.jit
def kernel(pillar_features, voxel_coords):
    pf = pillar_features.reshape(B * C * M)
    vc = voxel_coords.astype(jnp.int32).reshape(B * M * 3)
    return _pillar_scatter(pf, vc)


# speedup vs baseline: 12.7215x; 1.0003x over previous
import functools

import jax
import jax.numpy as jnp
from jax import lax
from jax.experimental import pallas as pl
from jax.experimental.pallas import tpu as pltpu
from jax.experimental.pallas import tpu_sc as plsc

NX = 432
NY = 496
C = 64
M = 12000
B = 4
NXY = NX * NY
ROWS_BIG = 32
ROWS_SMALL = 24
NBIG = 14
CELLS_BIG = ROWS_BIG * NX
CELLS_SMALL = ROWS_SMALL * NX
CHUNK = 2000
NCHUNK = M // CHUNK
GC = CHUNK // 16
CAP = M + 16


def _sc_body(pf_hbm, vc_hbm, out_hbm,
             cbuf0, cbuf1, winner, cjw, row0, row1, ob0, ob1, tmp16,
             sem0, sem1, rs0, rs1, cs0, cs1):
    core = lax.axis_index("c")
    sid = lax.axis_index("s")
    big = sid < NBIG
    lo = jnp.where(big, sid * CELLS_BIG,
                   NBIG * CELLS_BIG + (sid - NBIG) * CELLS_SMALL)
    ncell = jnp.where(big, CELLS_BIG, CELLS_SMALL)
    y0 = pl.multiple_of(lo // NX, 8)
    gr = lax.shift_right_logical(ncell, 4)
    iota = lax.iota(jnp.int32, 16)
    zeros16 = jnp.zeros((16,), jnp.int32)

    def out_dma(ob, b, cpair, sem, do_start, do_wait):
        @pl.when(big)
        def _():
            cp = pltpu.make_async_copy(
                ob,
                out_hbm.at[b, pl.ds(cpair, 2), pl.ds(y0, ROWS_BIG)], sem)
            if do_start:
                cp.start()
            if do_wait:
                cp.wait()

        @pl.when(jnp.logical_not(big))
        def _():
            cp = pltpu.make_async_copy(
                ob.at[:, pl.ds(0, ROWS_SMALL)],
                out_hbm.at[b, pl.ds(cpair, 2), pl.ds(y0, ROWS_SMALL)], sem)
            if do_start:
                cp.start()
            if do_wait:
                cp.wait()

    for bb in range(2):
        b = core * 2 + bb

        def fill_body(g, _):
            winner[pl.ds(g * 16, 16)] = jnp.full((16,), -1, jnp.int32)
            return 0

        lax.fori_loop(0, CELLS_BIG // 16, fill_body, 0, unroll=4)

        def coord_dma(ch, cb, cs):
            off = pl.multiple_of(b * (M * 3) + ch * (CHUNK * 3), 8)
            return pltpu.make_async_copy(
                vc_hbm.at[pl.ds(off, CHUNK * 3)], cb, cs)

        def row_dma(c, rb, rsem):
            poff = pl.multiple_of((b * C + c) * M, 8)
            return pltpu.make_async_copy(pf_hbm.at[pl.ds(poff, M)], rb, rsem)

        row_dma(0, row0, rs0).start()
        row_dma(1, row1, rs1).start()

        coord_dma(0, cbuf0, cs0).start()
        coord_dma(1, cbuf1, cs1).start()
        for chp in range(NCHUNK // 2):
            for kk, (cb, cs) in enumerate(((cbuf0, cs0), (cbuf1, cs1))):
                ch = chp * 2 + kk
                coord_dma(ch, cb, cs).wait()

                def g_body(g, _):
                    rows = (g * 16 + iota) * 3
                    z = plsc.load_gather(cb, [rows])
                    y = plsc.load_gather(cb, [rows + 1])
                    x = plsc.load_gather(cb, [rows + 2])
                    idx = z + y * NX + x
                    key = idx * 16 + iota
                    m = ch * CHUNK + g * 16 + iota
                    ks, vs = plsc.sort_key_val(key, m)
                    sidx = lax.shift_right_logical(ks, 4)
                    tmp16[:] = sidx
                    nxt = plsc.load_gather(tmp16, [jnp.minimum(iota + 1, 15)])
                    is_last = (sidx != nxt) | (iota == 15)
                    in_rng = (sidx >= lo) & (sidx < lo + ncell)
                    msk = is_last & in_rng
                    pos = jnp.clip(sidx - lo, 0, CELLS_BIG - 1)
                    plsc.store_scatter(winner, [pos], vs, mask=msk)
                    return 0

                lax.fori_loop(0, GC, g_body, 0, unroll=5)
                if ch + 2 < NCHUNK:
                    coord_dma(ch + 2, cb, cs).start()

        def comp_body(g, off):
            w = winner[pl.ds(g * 16, 16)]
            msk = w >= 0
            j = g * 16 + iota
            packed = lax.shift_left(j, 14) | jnp.maximum(w, 0)
            plsc.store_compressed(cjw.at[pl.ds(off, 16)], packed, mask=msk)
            return off + jnp.max(plsc.all_reduce_population_count(msk))

        K = lax.fori_loop(0, CELLS_BIG // 16, comp_body, jnp.int32(0),
                          unroll=4)
        nK = lax.shift_right_logical(K + 15, 4)

        def zero_body(r, _):
            zv = jnp.zeros((16,), jnp.float32)

            def zcol(cg, _):
                for ob in (ob0, ob1):
                    ob[0, r, pl.ds(cg * 16, 16)] = zv
                    ob[1, r, pl.ds(cg * 16, 16)] = zv
                return 0

            lax.fori_loop(0, NX // 16, zcol, 0)
            return 0

        lax.fori_loop(0, ROWS_BIG, zero_body, 0)

        def quad_body(q, _):
            for k, (ob, sem) in enumerate(((ob0, sem0), (ob1, sem1))):
                cpair = q * 4 + 2 * k

                @pl.when(q > 0)
                def _wait_prev():
                    out_dma(ob, b, cpair, sem, do_start=False, do_wait=True)

                for h, (rb, rsem) in enumerate(((row0, rs0), (row1, rs1))):
                    c = cpair + h
                    row_dma(c, rb, rsem).wait()

                    def s_body(g, _):
                        ii = g * 16 + iota
                        msk = ii < K
                        pk = cjw[pl.ds(g * 16, 16)]
                        ww = jnp.minimum(pk & 0x3FFF, M - 1)
                        jj = jnp.minimum(
                            lax.shift_right_logical(pk, 14), CELLS_BIG - 1)
                        vals = plsc.load_gather(rb, [ww], mask=msk)
                        rr = jj // NX
                        xx = jj - rr * NX
                        plsc.store_scatter(
                            ob, [zeros16 + h, rr, xx], vals, mask=msk)
                        return 0

                    lax.fori_loop(0, nK, s_body, 0)

                    @pl.when(c + 2 < C)
                    def _prefetch_next():
                        row_dma(c + 2, rb, rsem).start()

                out_dma(ob, b, cpair, sem, do_start=True, do_wait=False)
            return 0

        lax.fori_loop(0, C // 4, quad_body, 0)

        out_dma(ob0, b, 0, sem0, do_start=False, do_wait=True)
        out_dma(ob1, b, 0, sem1, do_start=False, do_wait=True)


@functools.partial(
    pl.kernel,
    out_type=jax.ShapeDtypeStruct((B, C, NY, NX), jnp.float32),
    mesh=plsc.VectorSubcoreMesh(core_axis_name="c", subcore_axis_name="s"),
    compiler_params=pltpu.CompilerParams(needs_layout_passes=False),
    scratch_types=[
        pltpu.VMEM((CHUNK * 3,), jnp.int32),
        pltpu.VMEM((CHUNK * 3,), jnp.int32),
        pltpu.VMEM((CELLS_BIG,), jnp.int32),
        pltpu.VMEM((CAP,), jnp.int32),
        pltpu.VMEM((M,), jnp.float32),
        pltpu.VMEM((M,), jnp.float32),
        pltpu.VMEM((2, ROWS_BIG, NX), jnp.float32),
        pltpu.VMEM((2, ROWS_BIG, NX), jnp.float32),
        pltpu.VMEM((16,), jnp.int32),
        pltpu.SemaphoreType.DMA,
        pltpu.SemaphoreType.DMA,
        pltpu.SemaphoreType.DMA,
        pltpu.SemaphoreType.DMA,
        pltpu.SemaphoreType.DMA,
        pltpu.SemaphoreType.DMA,
    ],
)
def _pillar_scatter(pf_hbm, vc_hbm, out_hbm, *scratch):
    _sc_body(pf_hbm, vc_hbm, out_hbm, *scratch)


@jax.jit
def kernel(pillar_features, voxel_coords):
    pf = pillar_features.reshape(B * C * M)
    vc = voxel_coords.astype(jnp.int32).reshape(B * M * 3)
    return _pillar_scatter(pf, vc)
